# baseline (speedup 1.0000x reference)
import functools

import jax
import jax.numpy as jnp
from jax import lax
from jax.experimental import pallas as pl
from jax.experimental.pallas import tpu as pltpu, tpu_sc as plsc

_BIG = 3.4e38


def _k1_body(c1n_ref, c2n_ref, c1_ref, c2_ref, idx_ref, wts_ref, *, n2, tn):
    b = pl.program_id(0)
    c1b = c1_ref[0]
    c2b = c2_ref[0]

    cross = jax.lax.dot_general(
        c2b, c1b, (((0,), (0,)), ((), ())),
        preferred_element_type=jnp.float32)
    dist = (c1n_ref[0] + c2n_ref[0]) - 2.0 * cross

    rows = jax.lax.broadcasted_iota(jnp.int32, (n2, tn), 0)
    d = dist
    recs = []
    idxs = []
    for k in range(3):
        m = jnp.min(d, axis=0, keepdims=True)
        idxk = jnp.min(jnp.where(d == m, rows, n2), axis=0, keepdims=True)
        recs.append(1.0 / (m + 1e-8))
        idxs.append(idxk)
        if k < 2:
            d = jnp.where(rows == idxk, _BIG, d)
    rsum = recs[0] + recs[1] + recs[2]

    idx_ref[0] = jnp.concatenate(idxs, axis=0) + b * n2
    wts_ref[0] = jnp.concatenate(
        [recs[0] / rsum, recs[1] / rsum, recs[2] / rsum], axis=0)


def _sc_interp_body(table_hbm, idx_hbm, w_hbm, out_hbm,
                    idx_v, w_v, r0, r1, r2, o_v, s0, s1, s2,
                    *, n1, c2, pc, wpb):
    wid = lax.axis_index("s") * 2 + lax.axis_index("c")
    b = wid // wpb
    part = wid % wpb
    span = n1 // wpb

    def chunk(c, _):
        base = part * span + c * pc
        for k3 in range(3):
            pltpu.sync_copy(idx_hbm.at[b, k3, pl.ds(base, pc)],
                            idx_v.at[pl.ds(k3 * pc, pc)])
            pltpu.sync_copy(w_hbm.at[b, k3, pl.ds(base, pc)],
                            w_v.at[pl.ds(k3 * pc, pc)])
        cp0 = pltpu.async_copy(table_hbm.at[idx_v.at[pl.ds(0, pc)]], r0, s0)
        cp1 = pltpu.async_copy(table_hbm.at[idx_v.at[pl.ds(pc, pc)]], r1, s1)
        cp2 = pltpu.async_copy(table_hbm.at[idx_v.at[pl.ds(2 * pc, pc)]], r2, s2)
        cp0.wait()
        cp1.wait()
        cp2.wait()

        def group(g, _):
            w0s = w_v[pl.ds(g * 16, 16)]
            w1s = w_v[pl.ds(pc + g * 16, 16)]
            w2s = w_v[pl.ds(2 * pc + g * 16, 16)]
            for pp in range(16):
                lane = jnp.full((16,), pp, jnp.int32)
                w0 = w0s[lane]
                w1 = w1s[lane]
                w2 = w2s[lane]
                p = g * 16 + pp
                for s in range(c2 // 16):
                    sl = pl.ds(s * 16, 16)
                    o_v[p, sl] = (w0 * r0[p, sl] + w1 * r1[p, sl]
                                  + w2 * r2[p, sl])
            return 0

        lax.fori_loop(0, pc // 16, group, 0)
        pltpu.sync_copy(o_v, out_hbm.at[b, pl.ds(base, pc)])
        return 0

    lax.fori_loop(0, span // pc, chunk, 0)


def _k2_body(f1_ref, it_ref, w1a_ref, w1b_ref, b1_ref,
             h1_ref, s1_ref, q1_ref):
    b = pl.program_id(0)
    i = pl.program_id(1)
    h1 = (jax.lax.dot_general(
              w1a_ref[...], f1_ref[0], (((1,), (0,)), ((), ())),
              preferred_element_type=jnp.float32)
          + jax.lax.dot_general(
              w1b_ref[...], it_ref[0], (((1,), (1,)), ((), ())),
              preferred_element_type=jnp.float32)
          + b1_ref[...])
    h1_ref[0] = h1

    @pl.when((b == 0) & (i == 0))
    def _():
        s1_ref[...] = jnp.zeros_like(s1_ref)
        q1_ref[...] = jnp.zeros_like(q1_ref)

    s1_ref[...] += jnp.sum(h1, axis=1, keepdims=True)
    q1_ref[...] += jnp.sum(h1 * h1, axis=1, keepdims=True)


def _bn_affine(s_ref, q_ref, g_ref, be_ref, count):
    mean = s_ref[...] / count
    var = q_ref[...] / count - mean * mean
    scale = g_ref[...] * jax.lax.rsqrt(var + 1e-5)
    shift = be_ref[...] - mean * scale
    return scale, shift


def _layer2(h1_ref, s1_ref, q1_ref, g1_ref, be1_ref, w2_ref, b2_ref, count):
    scale, shift = _bn_affine(s1_ref, q1_ref, g1_ref, be1_ref, count)
    a = jnp.maximum(h1_ref[0] * scale + shift, 0.0)
    return jax.lax.dot_general(
        w2_ref[...], a, (((1,), (0,)), ((), ())),
        preferred_element_type=jnp.float32) + b2_ref[...]


def _k3_body(h1_ref, s1_ref, q1_ref, g1_ref, be1_ref, w2_ref, b2_ref,
             s2_ref, q2_ref, *, count):
    b = pl.program_id(0)
    i = pl.program_id(1)
    h2 = _layer2(h1_ref, s1_ref, q1_ref, g1_ref, be1_ref, w2_ref, b2_ref,
                 count)

    @pl.when((b == 0) & (i == 0))
    def _():
        s2_ref[...] = jnp.zeros_like(s2_ref)
        q2_ref[...] = jnp.zeros_like(q2_ref)

    s2_ref[...] += jnp.sum(h2, axis=1, keepdims=True)
    q2_ref[...] += jnp.sum(h2 * h2, axis=1, keepdims=True)


def _k4_body(h1_ref, s1_ref, q1_ref, g1_ref, be1_ref, w2_ref, b2_ref,
             s2_ref, q2_ref, g2_ref, be2_ref, o_ref, *, count):
    h2 = _layer2(h1_ref, s1_ref, q1_ref, g1_ref, be1_ref, w2_ref, b2_ref,
                 count)
    scale, shift = _bn_affine(s2_ref, q2_ref, g2_ref, be2_ref, count)
    o_ref[0] = jnp.maximum(h2 * scale + shift, 0.0)


def kernel(centroids1, centroids2, features1, features2,
           W1, b1, g1, be1, W2, b2, g2, be2):
    B, _, N1 = centroids1.shape
    N2 = centroids2.shape[2]
    C1 = features1.shape[1]
    C2 = features2.shape[1]
    CO = W1.shape[0]
    TN = min(512, N1)
    NB = N1 // TN
    count = float(B * N1)
    NW = 32
    WPB = NW // B
    PC = 64

    b1c = b1.reshape(CO, 1)
    g1c = g1.reshape(CO, 1)
    be1c = be1.reshape(CO, 1)
    b2c = b2.reshape(CO, 1)
    g2c = g2.reshape(CO, 1)
    be2c = be2.reshape(CO, 1)
    W1a = W1[:, :C1]
    W1b = W1[:, C1:]

    grid = (B, NB)
    col_spec = pl.BlockSpec((CO, 1), lambda b, i: (0, 0))

    c1n = jnp.sum(jnp.transpose(centroids1, (0, 2, 1)) ** 2,
                  axis=-1).reshape(B, 1, N1)
    c2n = jnp.sum(jnp.transpose(centroids2, (0, 2, 1)) ** 2,
                  axis=-1).reshape(B, N2, 1)
    table = jnp.transpose(features2, (0, 2, 1)).reshape(B * N2, C2)

    idxg, wts = pl.pallas_call(
        functools.partial(_k1_body, n2=N2, tn=TN),
        grid=grid,
        in_specs=[
            pl.BlockSpec((1, 1, TN), lambda b, i: (b, 0, i)),
            pl.BlockSpec((1, N2, 1), lambda b, i: (b, 0, 0)),
            pl.BlockSpec((1, 3, TN), lambda b, i: (b, 0, i)),
            pl.BlockSpec((1, 3, N2), lambda b, i: (b, 0, 0)),
        ],
        out_specs=[
            pl.BlockSpec((1, 3, TN), lambda b, i: (b, 0, i)),
            pl.BlockSpec((1, 3, TN), lambda b, i: (b, 0, i)),
        ],
        out_shape=[
            jax.ShapeDtypeStruct((B, 3, N1), jnp.int32),
            jax.ShapeDtypeStruct((B, 3, N1), jnp.float32),
        ],
    )(c1n, c2n, centroids1, centroids2)

    mesh = plsc.VectorSubcoreMesh(core_axis_name="c", subcore_axis_name="s")
    interp = functools.partial(
        pl.kernel,
        mesh=mesh,
        out_type=jax.ShapeDtypeStruct((B, N1, C2), jnp.float32),
        scratch_types=[
            pltpu.VMEM((3 * PC,), jnp.int32),
            pltpu.VMEM((3 * PC,), jnp.float32),
            pltpu.VMEM((PC, C2), jnp.float32),
            pltpu.VMEM((PC, C2), jnp.float32),
            pltpu.VMEM((PC, C2), jnp.float32),
            pltpu.VMEM((PC, C2), jnp.float32),
            pltpu.SemaphoreType.DMA,
            pltpu.SemaphoreType.DMA,
            pltpu.SemaphoreType.DMA,
        ],
    )(functools.partial(_sc_interp_body, n1=N1, c2=C2, pc=PC, wpb=WPB)
      )(table, idxg, wts)

    h1pre, s1, q1 = pl.pallas_call(
        _k2_body,
        grid=grid,
        in_specs=[
            pl.BlockSpec((1, C1, TN), lambda b, i: (b, 0, i)),
            pl.BlockSpec((1, TN, C2), lambda b, i: (b, i, 0)),
            pl.BlockSpec((CO, C1), lambda b, i: (0, 0)),
            pl.BlockSpec((CO, C2), lambda b, i: (0, 0)),
            col_spec,
        ],
        out_specs=[
            pl.BlockSpec((1, CO, TN), lambda b, i: (b, 0, i)),
            col_spec,
            col_spec,
        ],
        out_shape=[
            jax.ShapeDtypeStruct((B, CO, N1), jnp.float32),
            jax.ShapeDtypeStruct((CO, 1), jnp.float32),
            jax.ShapeDtypeStruct((CO, 1), jnp.float32),
        ],
    )(features1, interp, W1a, W1b, b1c)

    TM = min(1024, N1)
    grid2 = (B, N1 // TM)
    blk = pl.BlockSpec((1, CO, TM), lambda b, i: (b, 0, i))
    w2_spec = pl.BlockSpec((CO, CO), lambda b, i: (0, 0))

    s2, q2 = pl.pallas_call(
        functools.partial(_k3_body, count=count),
        grid=grid2,
        in_specs=[blk, col_spec, col_spec, col_spec, col_spec, w2_spec,
                  col_spec],
        out_specs=[col_spec, col_spec],
        out_shape=[
            jax.ShapeDtypeStruct((CO, 1), jnp.float32),
            jax.ShapeDtypeStruct((CO, 1), jnp.float32),
        ],
    )(h1pre, s1, q1, g1c, be1c, W2, b2c)

    out = pl.pallas_call(
        functools.partial(_k4_body, count=count),
        grid=grid2,
        in_specs=[blk, col_spec, col_spec, col_spec, col_spec, w2_spec,
                  col_spec, col_spec, col_spec, col_spec, col_spec],
        out_specs=blk,
        out_shape=jax.ShapeDtypeStruct((B, CO, N1), jnp.float32),
    )(h1pre, s1, q1, g1c, be1c, W2, b2c, s2, q2, g2c, be2c)

    return out


# speedup vs baseline: 1.0683x; 1.0683x over previous
import functools

import jax
import jax.numpy as jnp
from jax import lax
from jax.experimental import pallas as pl
from jax.experimental.pallas import tpu as pltpu, tpu_sc as plsc

_BIG = 3.4e38


def _k1_body(c1n_ref, c2n_ref, c1_ref, c2_ref, idx_ref, wts_ref, *, n2, tn):
    b = pl.program_id(0)
    c1b = c1_ref[0]
    c2b = c2_ref[0]

    cross = jax.lax.dot_general(
        c2b, c1b, (((0,), (0,)), ((), ())),
        preferred_element_type=jnp.float32)
    dist = (c1n_ref[0] + c2n_ref[0]) - 2.0 * cross

    rows = jax.lax.broadcasted_iota(jnp.int32, (n2, tn), 0)
    d = dist
    recs = []
    idxs = []
    for k in range(3):
        m = jnp.min(d, axis=0, keepdims=True)
        idxk = jnp.min(jnp.where(d == m, rows, n2), axis=0, keepdims=True)
        recs.append(1.0 / (m + 1e-8))
        idxs.append(idxk)
        if k < 2:
            d = jnp.where(rows == idxk, _BIG, d)
    rsum = recs[0] + recs[1] + recs[2]

    idx_ref[0] = jnp.concatenate(idxs, axis=0) + b * n2
    wts_ref[0] = jnp.concatenate(
        [recs[0] / rsum, recs[1] / rsum, recs[2] / rsum], axis=0)


def _sc_interp_body(table_hbm, idx_hbm, w_hbm, out_hbm,
                    idx_v, w_v, r00, r01, r02, r10, r11, r12, o0, o1,
                    g00, g01, g02, g10, g11, g12, so0, so1,
                    *, n1, c2, pc, wpb):
    wid = lax.axis_index("s") * 2 + lax.axis_index("c")
    b = wid // wpb
    part = wid % wpb
    span = n1 // wpb
    start = part * span
    nchunks = span // pc
    rbufs = ((r00, r01, r02), (r10, r11, r12))
    gsems = ((g00, g01, g02), (g10, g11, g12))
    obufs = (o0, o1)
    osems = (so0, so1)

    pltpu.sync_copy(idx_hbm.at[b, :, pl.ds(start, span)], idx_v)
    pltpu.sync_copy(w_hbm.at[b, :, pl.ds(start, span)], w_v)

    def fire(c, buf):
        for k3 in range(3):
            pltpu.async_copy(
                table_hbm.at[idx_v.at[k3, pl.ds(c * pc, pc)]],
                rbufs[buf][k3], gsems[buf][k3])

    def wait_gathers(c, buf):
        for k3 in range(3):
            pltpu.make_async_copy(
                table_hbm.at[idx_v.at[k3, pl.ds(c * pc, pc)]],
                rbufs[buf][k3], gsems[buf][k3]).wait()

    def compute(c, buf, t):
        r0, r1, r2 = rbufs[buf]
        ov = obufs[buf]

        @pl.when(t > 0)
        def _():
            pltpu.make_async_copy(
                ov, out_hbm.at[b, pl.ds(0, pc)], osems[buf]).wait()

        def group(g, _):
            w0s = w_v[0, pl.ds(c * pc + g * 16, 16)]
            w1s = w_v[1, pl.ds(c * pc + g * 16, 16)]
            w2s = w_v[2, pl.ds(c * pc + g * 16, 16)]
            for pp in range(16):
                lane = jnp.full((16,), pp, jnp.int32)
                w0 = w0s[lane]
                w1 = w1s[lane]
                w2 = w2s[lane]
                p = g * 16 + pp
                for s in range(c2 // 16):
                    sl = pl.ds(s * 16, 16)
                    ov[p, sl] = (w0 * r0[p, sl] + w1 * r1[p, sl]
                                 + w2 * r2[p, sl])
            return 0

        lax.fori_loop(0, pc // 16, group, 0)
        pltpu.async_copy(ov, out_hbm.at[b, pl.ds(start + c * pc, pc)],
                         osems[buf])

    fire(0, 0)

    def body(t, _):
        c0 = 2 * t
        fire(c0 + 1, 1)
        wait_gathers(c0, 0)
        compute(c0, 0, t)

        @pl.when(c0 + 2 < nchunks)
        def _():
            fire(c0 + 2, 0)

        wait_gathers(c0 + 1, 1)
        compute(c0 + 1, 1, t)
        return 0

    lax.fori_loop(0, nchunks // 2, body, 0)
    for buf in range(2):
        pltpu.make_async_copy(
            obufs[buf], out_hbm.at[b, pl.ds(0, pc)], osems[buf]).wait()


def _k2_body(f1_ref, it_ref, w1a_ref, w1b_ref, b1_ref,
             h1_ref, s1_ref, q1_ref):
    b = pl.program_id(0)
    i = pl.program_id(1)
    h1 = (jax.lax.dot_general(
              w1a_ref[...], f1_ref[0], (((1,), (0,)), ((), ())),
              preferred_element_type=jnp.float32)
          + jax.lax.dot_general(
              w1b_ref[...], it_ref[0], (((1,), (1,)), ((), ())),
              preferred_element_type=jnp.float32)
          + b1_ref[...])
    h1_ref[0] = h1

    @pl.when((b == 0) & (i == 0))
    def _():
        s1_ref[...] = jnp.zeros_like(s1_ref)
        q1_ref[...] = jnp.zeros_like(q1_ref)

    s1_ref[...] += jnp.sum(h1, axis=1, keepdims=True)
    q1_ref[...] += jnp.sum(h1 * h1, axis=1, keepdims=True)


def _bn_affine(s_ref, q_ref, g_ref, be_ref, count):
    mean = s_ref[...] / count
    var = q_ref[...] / count - mean * mean
    scale = g_ref[...] * jax.lax.rsqrt(var + 1e-5)
    shift = be_ref[...] - mean * scale
    return scale, shift


def _layer2(h1_ref, s1_ref, q1_ref, g1_ref, be1_ref, w2_ref, b2_ref, count):
    scale, shift = _bn_affine(s1_ref, q1_ref, g1_ref, be1_ref, count)
    a = jnp.maximum(h1_ref[0] * scale + shift, 0.0)
    return jax.lax.dot_general(
        w2_ref[...], a, (((1,), (0,)), ((), ())),
        preferred_element_type=jnp.float32) + b2_ref[...]


def _k3_body(h1_ref, s1_ref, q1_ref, g1_ref, be1_ref, w2_ref, b2_ref,
             s2_ref, q2_ref, *, count):
    b = pl.program_id(0)
    i = pl.program_id(1)
    h2 = _layer2(h1_ref, s1_ref, q1_ref, g1_ref, be1_ref, w2_ref, b2_ref,
                 count)

    @pl.when((b == 0) & (i == 0))
    def _():
        s2_ref[...] = jnp.zeros_like(s2_ref)
        q2_ref[...] = jnp.zeros_like(q2_ref)

    s2_ref[...] += jnp.sum(h2, axis=1, keepdims=True)
    q2_ref[...] += jnp.sum(h2 * h2, axis=1, keepdims=True)


def _k4_body(h1_ref, s1_ref, q1_ref, g1_ref, be1_ref, w2_ref, b2_ref,
             s2_ref, q2_ref, g2_ref, be2_ref, o_ref, *, count):
    h2 = _layer2(h1_ref, s1_ref, q1_ref, g1_ref, be1_ref, w2_ref, b2_ref,
                 count)
    scale, shift = _bn_affine(s2_ref, q2_ref, g2_ref, be2_ref, count)
    o_ref[0] = jnp.maximum(h2 * scale + shift, 0.0)


def kernel(centroids1, centroids2, features1, features2,
           W1, b1, g1, be1, W2, b2, g2, be2):
    B, _, N1 = centroids1.shape
    N2 = centroids2.shape[2]
    C1 = features1.shape[1]
    C2 = features2.shape[1]
    CO = W1.shape[0]
    TN = min(512, N1)
    NB = N1 // TN
    count = float(B * N1)
    NW = 32
    WPB = NW // B
    SPAN = N1 // WPB
    PC = 32

    b1c = b1.reshape(CO, 1)
    g1c = g1.reshape(CO, 1)
    be1c = be1.reshape(CO, 1)
    b2c = b2.reshape(CO, 1)
    g2c = g2.reshape(CO, 1)
    be2c = be2.reshape(CO, 1)
    W1a = W1[:, :C1]
    W1b = W1[:, C1:]

    grid = (B, NB)
    col_spec = pl.BlockSpec((CO, 1), lambda b, i: (0, 0))

    c1n = jnp.sum(jnp.transpose(centroids1, (0, 2, 1)) ** 2,
                  axis=-1).reshape(B, 1, N1)
    c2n = jnp.sum(jnp.transpose(centroids2, (0, 2, 1)) ** 2,
                  axis=-1).reshape(B, N2, 1)
    table = jnp.transpose(features2, (0, 2, 1)).reshape(B * N2, C2)

    idxg, wts = pl.pallas_call(
        functools.partial(_k1_body, n2=N2, tn=TN),
        grid=grid,
        in_specs=[
            pl.BlockSpec((1, 1, TN), lambda b, i: (b, 0, i)),
            pl.BlockSpec((1, N2, 1), lambda b, i: (b, 0, 0)),
            pl.BlockSpec((1, 3, TN), lambda b, i: (b, 0, i)),
            pl.BlockSpec((1, 3, N2), lambda b, i: (b, 0, 0)),
        ],
        out_specs=[
            pl.BlockSpec((1, 3, TN), lambda b, i: (b, 0, i)),
            pl.BlockSpec((1, 3, TN), lambda b, i: (b, 0, i)),
        ],
        out_shape=[
            jax.ShapeDtypeStruct((B, 3, N1), jnp.int32),
            jax.ShapeDtypeStruct((B, 3, N1), jnp.float32),
        ],
    )(c1n, c2n, centroids1, centroids2)

    mesh = plsc.VectorSubcoreMesh(core_axis_name="c", subcore_axis_name="s")
    interp = functools.partial(
        pl.kernel,
        mesh=mesh,
        out_type=jax.ShapeDtypeStruct((B, N1, C2), jnp.float32),
        scratch_types=(
            [pltpu.VMEM((3, SPAN), jnp.int32),
             pltpu.VMEM((3, SPAN), jnp.float32)]
            + [pltpu.VMEM((PC, C2), jnp.float32) for _ in range(8)]
            + [pltpu.SemaphoreType.DMA for _ in range(8)]
        ),
    )(functools.partial(_sc_interp_body, n1=N1, c2=C2, pc=PC, wpb=WPB)
      )(table, idxg, wts)

    h1pre, s1, q1 = pl.pallas_call(
        _k2_body,
        grid=grid,
        in_specs=[
            pl.BlockSpec((1, C1, TN), lambda b, i: (b, 0, i)),
            pl.BlockSpec((1, TN, C2), lambda b, i: (b, i, 0)),
            pl.BlockSpec((CO, C1), lambda b, i: (0, 0)),
            pl.BlockSpec((CO, C2), lambda b, i: (0, 0)),
            col_spec,
        ],
        out_specs=[
            pl.BlockSpec((1, CO, TN), lambda b, i: (b, 0, i)),
            col_spec,
            col_spec,
        ],
        out_shape=[
            jax.ShapeDtypeStruct((B, CO, N1), jnp.float32),
            jax.ShapeDtypeStruct((CO, 1), jnp.float32),
            jax.ShapeDtypeStruct((CO, 1), jnp.float32),
        ],
    )(features1, interp, W1a, W1b, b1c)

    TM = min(1024, N1)
    grid2 = (B, N1 // TM)
    blk = pl.BlockSpec((1, CO, TM), lambda b, i: (b, 0, i))
    w2_spec = pl.BlockSpec((CO, CO), lambda b, i: (0, 0))

    s2, q2 = pl.pallas_call(
        functools.partial(_k3_body, count=count),
        grid=grid2,
        in_specs=[blk, col_spec, col_spec, col_spec, col_spec, w2_spec,
                  col_spec],
        out_specs=[col_spec, col_spec],
        out_shape=[
            jax.ShapeDtypeStruct((CO, 1), jnp.float32),
            jax.ShapeDtypeStruct((CO, 1), jnp.float32),
        ],
    )(h1pre, s1, q1, g1c, be1c, W2, b2c)

    out = pl.pallas_call(
        functools.partial(_k4_body, count=count),
        grid=grid2,
        in_specs=[blk, col_spec, col_spec, col_spec, col_spec, w2_spec,
                  col_spec, col_spec, col_spec, col_spec, col_spec],
        out_specs=blk,
        out_shape=jax.ShapeDtypeStruct((B, CO, N1), jnp.float32),
    )(h1pre, s1, q1, g1c, be1c, W2, b2c, s2, q2, g2c, be2c)

    return out


# speedup vs baseline: 1.2741x; 1.1926x over previous
import functools

import jax
import jax.numpy as jnp
from jax import lax
from jax.experimental import pallas as pl
from jax.experimental.pallas import tpu as pltpu, tpu_sc as plsc

_BIG = 3.4e38


def _k1_body(c1n_ref, c2n_ref, c1_ref, c2_ref, idx_ref, wts_ref, *, n2, tn,
             boffs=0):
    b = pl.program_id(0)
    c1b = c1_ref[0]
    c2b = c2_ref[0]

    cross = jax.lax.dot_general(
        c2b, c1b, (((0,), (0,)), ((), ())),
        preferred_element_type=jnp.float32)
    dist = (c1n_ref[0] + c2n_ref[0]) - 2.0 * cross

    rows = jax.lax.broadcasted_iota(jnp.int32, (n2, tn), 0)
    d = dist
    recs = []
    idxs = []
    for k in range(3):
        m = jnp.min(d, axis=0, keepdims=True)
        idxk = jnp.min(jnp.where(d == m, rows, n2), axis=0, keepdims=True)
        recs.append(1.0 / (m + 1e-8))
        idxs.append(idxk)
        if k < 2:
            d = jnp.where(rows == idxk, _BIG, d)
    rsum = recs[0] + recs[1] + recs[2]

    idx_ref[0] = jnp.concatenate(idxs, axis=0) + (b + boffs) * n2
    wts_ref[0] = jnp.concatenate(
        [recs[0] / rsum, recs[1] / rsum, recs[2] / rsum], axis=0)


def _sc_interp_body(table_hbm, idx_hbm, w_hbm, out_hbm,
                    idx_v, w_v, r00, r01, r02, r10, r11, r12, o0, o1,
                    g00, g01, g02, g10, g11, g12, so0, so1,
                    *, n1, c2, pc, wpb):
    wid = lax.axis_index("s") * 2 + lax.axis_index("c")
    b = wid // wpb
    part = wid % wpb
    span = n1 // wpb
    start = part * span
    nchunks = span // pc
    rbufs = ((r00, r01, r02), (r10, r11, r12))
    gsems = ((g00, g01, g02), (g10, g11, g12))
    obufs = (o0, o1)
    osems = (so0, so1)

    pltpu.sync_copy(idx_hbm.at[b, :, pl.ds(start, span)], idx_v)
    pltpu.sync_copy(w_hbm.at[b, :, pl.ds(start, span)], w_v)

    def fire(c, buf):
        for k3 in range(3):
            pltpu.async_copy(
                table_hbm.at[idx_v.at[k3, pl.ds(c * pc, pc)]],
                rbufs[buf][k3], gsems[buf][k3])

    def wait_gathers(c, buf):
        for k3 in range(3):
            pltpu.make_async_copy(
                table_hbm.at[idx_v.at[k3, pl.ds(c * pc, pc)]],
                rbufs[buf][k3], gsems[buf][k3]).wait()

    def compute(c, buf, t):
        r0, r1, r2 = rbufs[buf]
        ov = obufs[buf]

        @pl.when(t > 0)
        def _():
            pltpu.make_async_copy(
                ov, out_hbm.at[b, pl.ds(0, pc)], osems[buf]).wait()

        def group(g, _):
            w0s = w_v[0, pl.ds(c * pc + g * 16, 16)]
            w1s = w_v[1, pl.ds(c * pc + g * 16, 16)]
            w2s = w_v[2, pl.ds(c * pc + g * 16, 16)]
            for pp in range(16):
                lane = jnp.full((16,), pp, jnp.int32)
                w0 = w0s[lane]
                w1 = w1s[lane]
                w2 = w2s[lane]
                p = g * 16 + pp
                for s in range(c2 // 16):
                    sl = pl.ds(s * 16, 16)
                    ov[p, sl] = (w0 * r0[p, sl] + w1 * r1[p, sl]
                                 + w2 * r2[p, sl])
            return 0

        lax.fori_loop(0, pc // 16, group, 0)
        pltpu.async_copy(ov, out_hbm.at[b, pl.ds(start + c * pc, pc)],
                         osems[buf])

    fire(0, 0)

    def body(t, _):
        c0 = 2 * t
        fire(c0 + 1, 1)
        wait_gathers(c0, 0)
        compute(c0, 0, t)

        @pl.when(c0 + 2 < nchunks)
        def _():
            fire(c0 + 2, 0)

        wait_gathers(c0 + 1, 1)
        compute(c0 + 1, 1, t)
        return 0

    lax.fori_loop(0, nchunks // 2, body, 0)
    for buf in range(2):
        pltpu.make_async_copy(
            obufs[buf], out_hbm.at[b, pl.ds(0, pc)], osems[buf]).wait()


def _k2_body(f1_ref, it_ref, w1a_ref, w1b_ref, b1_ref, s1i_ref, q1i_ref,
             h1_ref, s1_ref, q1_ref):
    b = pl.program_id(0)
    i = pl.program_id(1)
    h1 = (jax.lax.dot_general(
              w1a_ref[...], f1_ref[0], (((1,), (0,)), ((), ())),
              preferred_element_type=jnp.float32)
          + jax.lax.dot_general(
              w1b_ref[...], it_ref[0], (((1,), (1,)), ((), ())),
              preferred_element_type=jnp.float32)
          + b1_ref[...])
    h1_ref[0] = h1

    @pl.when((b == 0) & (i == 0))
    def _():
        s1_ref[...] = s1i_ref[...]
        q1_ref[...] = q1i_ref[...]

    s1_ref[...] += jnp.sum(h1, axis=1, keepdims=True)
    q1_ref[...] += jnp.sum(h1 * h1, axis=1, keepdims=True)


def _bn_affine(s_ref, q_ref, g_ref, be_ref, count):
    mean = s_ref[...] / count
    var = q_ref[...] / count - mean * mean
    scale = g_ref[...] * jax.lax.rsqrt(var + 1e-5)
    shift = be_ref[...] - mean * scale
    return scale, shift


def _layer2(h1_ref, s1_ref, q1_ref, g1_ref, be1_ref, w2_ref, b2_ref, count):
    scale, shift = _bn_affine(s1_ref, q1_ref, g1_ref, be1_ref, count)
    a = jnp.maximum(h1_ref[0] * scale + shift, 0.0)
    return jax.lax.dot_general(
        w2_ref[...], a, (((1,), (0,)), ((), ())),
        preferred_element_type=jnp.float32) + b2_ref[...]


def _k3_body(h1_ref, s1_ref, q1_ref, g1_ref, be1_ref, w2_ref, b2_ref,
             s2i_ref, q2i_ref, s2_ref, q2_ref, *, count):
    b = pl.program_id(0)
    i = pl.program_id(1)
    h2 = _layer2(h1_ref, s1_ref, q1_ref, g1_ref, be1_ref, w2_ref, b2_ref,
                 count)

    @pl.when((b == 0) & (i == 0))
    def _():
        s2_ref[...] = s2i_ref[...]
        q2_ref[...] = q2i_ref[...]

    s2_ref[...] += jnp.sum(h2, axis=1, keepdims=True)
    q2_ref[...] += jnp.sum(h2 * h2, axis=1, keepdims=True)


def _k4_body(h1_ref, s1_ref, q1_ref, g1_ref, be1_ref, w2_ref, b2_ref,
             s2_ref, q2_ref, g2_ref, be2_ref, o_ref, *, count):
    h2 = _layer2(h1_ref, s1_ref, q1_ref, g1_ref, be1_ref, w2_ref, b2_ref,
                 count)
    scale, shift = _bn_affine(s2_ref, q2_ref, g2_ref, be2_ref, count)
    o_ref[0] = jnp.maximum(h2 * scale + shift, 0.0)


def kernel(centroids1, centroids2, features1, features2,
           W1, b1, g1, be1, W2, b2, g2, be2):
    B, _, N1 = centroids1.shape
    N2 = centroids2.shape[2]
    C1 = features1.shape[1]
    C2 = features2.shape[1]
    CO = W1.shape[0]
    TN = min(512, N1)
    NB = N1 // TN
    count = float(B * N1)
    NW = 32
    WPB = NW // B
    SPAN = N1 // WPB
    PC = 32

    b1c = b1.reshape(CO, 1)
    g1c = g1.reshape(CO, 1)
    be1c = be1.reshape(CO, 1)
    b2c = b2.reshape(CO, 1)
    g2c = g2.reshape(CO, 1)
    be2c = be2.reshape(CO, 1)
    W1a = W1[:, :C1]
    W1b = W1[:, C1:]

    grid = (B, NB)
    col_spec = pl.BlockSpec((CO, 1), lambda b, i: (0, 0))

    c1n = jnp.sum(jnp.transpose(centroids1, (0, 2, 1)) ** 2,
                  axis=-1).reshape(B, 1, N1)
    c2n = jnp.sum(jnp.transpose(centroids2, (0, 2, 1)) ** 2,
                  axis=-1).reshape(B, N2, 1)
    table = jnp.transpose(features2, (0, 2, 1)).reshape(B * N2, C2)

    B2 = B // 2
    WPB2 = NW // B2
    SPAN2 = N1 // WPB2
    grid_h = (B2, NB)
    mesh = plsc.VectorSubcoreMesh(core_axis_name="c", subcore_axis_name="s")

    idxw = []
    for h in range(2):
        sl = slice(h * B2, (h + 1) * B2)
        idxw.append(pl.pallas_call(
            functools.partial(_k1_body, n2=N2, tn=TN, boffs=h * B2),
            grid=grid_h,
            in_specs=[
                pl.BlockSpec((1, 1, TN), lambda b, i: (b, 0, i)),
                pl.BlockSpec((1, N2, 1), lambda b, i: (b, 0, 0)),
                pl.BlockSpec((1, 3, TN), lambda b, i: (b, 0, i)),
                pl.BlockSpec((1, 3, N2), lambda b, i: (b, 0, 0)),
            ],
            out_specs=[
                pl.BlockSpec((1, 3, TN), lambda b, i: (b, 0, i)),
                pl.BlockSpec((1, 3, TN), lambda b, i: (b, 0, i)),
            ],
            out_shape=[
                jax.ShapeDtypeStruct((B2, 3, N1), jnp.int32),
                jax.ShapeDtypeStruct((B2, 3, N1), jnp.float32),
            ],
        )(c1n[sl], c2n[sl], centroids1[sl], centroids2[sl]))

    interps = []
    for h in range(2):
        idxg, wts = idxw[h]
        interps.append(functools.partial(
            pl.kernel,
            mesh=mesh,
            out_type=jax.ShapeDtypeStruct((B2, N1, C2), jnp.float32),
            scratch_types=(
                [pltpu.VMEM((3, SPAN2), jnp.int32),
                 pltpu.VMEM((3, SPAN2), jnp.float32)]
                + [pltpu.VMEM((PC, C2), jnp.float32) for _ in range(8)]
                + [pltpu.SemaphoreType.DMA for _ in range(8)]
            ),
        )(functools.partial(_sc_interp_body, n1=N1, c2=C2, pc=PC, wpb=WPB2)
          )(table, idxg, wts))

    zcol = jnp.zeros((CO, 1), jnp.float32)
    s1, q1 = zcol, zcol
    h1pres = []
    for h in range(2):
        sl = slice(h * B2, (h + 1) * B2)
        h1pre, s1, q1 = pl.pallas_call(
            _k2_body,
            grid=grid_h,
            in_specs=[
                pl.BlockSpec((1, C1, TN), lambda b, i: (b, 0, i)),
                pl.BlockSpec((1, TN, C2), lambda b, i: (b, i, 0)),
                pl.BlockSpec((CO, C1), lambda b, i: (0, 0)),
                pl.BlockSpec((CO, C2), lambda b, i: (0, 0)),
                col_spec, col_spec, col_spec,
            ],
            out_specs=[
                pl.BlockSpec((1, CO, TN), lambda b, i: (b, 0, i)),
                col_spec,
                col_spec,
            ],
            out_shape=[
                jax.ShapeDtypeStruct((B2, CO, N1), jnp.float32),
                jax.ShapeDtypeStruct((CO, 1), jnp.float32),
                jax.ShapeDtypeStruct((CO, 1), jnp.float32),
            ],
        )(features1[sl], interps[h], W1a, W1b, b1c, s1, q1)
        h1pres.append(h1pre)

    TM = min(1024, N1)
    grid2h = (B2, N1 // TM)
    blk = pl.BlockSpec((1, CO, TM), lambda b, i: (b, 0, i))
    w2_spec = pl.BlockSpec((CO, CO), lambda b, i: (0, 0))

    s2, q2 = zcol, zcol
    for h in range(2):
        s2, q2 = pl.pallas_call(
            functools.partial(_k3_body, count=count),
            grid=grid2h,
            in_specs=[blk, col_spec, col_spec, col_spec, col_spec, w2_spec,
                      col_spec, col_spec, col_spec],
            out_specs=[col_spec, col_spec],
            out_shape=[
                jax.ShapeDtypeStruct((CO, 1), jnp.float32),
                jax.ShapeDtypeStruct((CO, 1), jnp.float32),
            ],
        )(h1pres[h], s1, q1, g1c, be1c, W2, b2c, s2, q2)

    outs = []
    for h in range(2):
        outs.append(pl.pallas_call(
            functools.partial(_k4_body, count=count),
            grid=grid2h,
            in_specs=[blk, col_spec, col_spec, col_spec, col_spec, w2_spec,
                      col_spec, col_spec, col_spec, col_spec, col_spec],
            out_specs=blk,
            out_shape=jax.ShapeDtypeStruct((B2, CO, N1), jnp.float32),
        )(h1pres[h], s1, q1, g1c, be1c, W2, b2c, s2, q2, g2c, be2c))

    return jnp.concatenate(outs, axis=0)


# speedup vs baseline: 1.3791x; 1.0824x over previous
import functools

import jax
import jax.numpy as jnp
from jax import lax
from jax.experimental import pallas as pl
from jax.experimental.pallas import tpu as pltpu, tpu_sc as plsc

_BIG = 3.4e38


def _k1_body(c1n_ref, c2n_ref, c1_ref, c2_ref, idx_ref, wts_ref, *, n2, tn,
             boffs=0):
    b = pl.program_id(0)
    c1b = c1_ref[0]
    c2b = c2_ref[0]

    cross = jax.lax.dot_general(
        c2b, c1b, (((0,), (0,)), ((), ())),
        preferred_element_type=jnp.float32)
    dist = (c1n_ref[0] + c2n_ref[0]) - 2.0 * cross

    rows = jax.lax.broadcasted_iota(jnp.int32, (n2, tn), 0)
    d = dist
    recs = []
    idxs = []
    for k in range(3):
        m = jnp.min(d, axis=0, keepdims=True)
        idxk = jnp.min(jnp.where(d == m, rows, n2), axis=0, keepdims=True)
        recs.append(1.0 / (m + 1e-8))
        idxs.append(idxk)
        if k < 2:
            d = jnp.where(rows == idxk, _BIG, d)
    rsum = recs[0] + recs[1] + recs[2]

    idx_ref[0] = jnp.concatenate(idxs, axis=0) + (b + boffs) * n2
    wts_ref[0] = jnp.concatenate(
        [recs[0] / rsum, recs[1] / rsum, recs[2] / rsum], axis=0)


def _sc_interp_body(table_hbm, idx_hbm, w_hbm, out_hbm,
                    idx_v, w_v, r00, r01, r02, r10, r11, r12, o0, o1,
                    g00, g01, g02, g10, g11, g12, so0, so1,
                    *, n1, c2, pc, wpb):
    wid = lax.axis_index("s") * 2 + lax.axis_index("c")
    b = wid // wpb
    part = wid % wpb
    span = n1 // wpb
    start = part * span
    nchunks = span // pc
    rbufs = ((r00, r01, r02), (r10, r11, r12))
    gsems = ((g00, g01, g02), (g10, g11, g12))
    obufs = (o0, o1)
    osems = (so0, so1)

    pltpu.sync_copy(idx_hbm.at[b, :, pl.ds(start, span)], idx_v)
    pltpu.sync_copy(w_hbm.at[b, :, pl.ds(start, span)], w_v)

    def fire(c, buf):
        for k3 in range(3):
            pltpu.async_copy(
                table_hbm.at[idx_v.at[k3, pl.ds(c * pc, pc)]],
                rbufs[buf][k3], gsems[buf][k3])

    def wait_gathers(c, buf):
        for k3 in range(3):
            pltpu.make_async_copy(
                table_hbm.at[idx_v.at[k3, pl.ds(c * pc, pc)]],
                rbufs[buf][k3], gsems[buf][k3]).wait()

    def compute(c, buf, t):
        r0, r1, r2 = rbufs[buf]
        ov = obufs[buf]

        @pl.when(t > 0)
        def _():
            pltpu.make_async_copy(
                ov, out_hbm.at[b, pl.ds(0, pc)], osems[buf]).wait()

        def group(g, _):
            w0s = w_v[0, pl.ds(c * pc + g * 16, 16)]
            w1s = w_v[1, pl.ds(c * pc + g * 16, 16)]
            w2s = w_v[2, pl.ds(c * pc + g * 16, 16)]
            for pp in range(16):
                lane = jnp.full((16,), pp, jnp.int32)
                w0 = w0s[lane]
                w1 = w1s[lane]
                w2 = w2s[lane]
                p = g * 16 + pp
                for s in range(c2 // 16):
                    sl = pl.ds(s * 16, 16)
                    ov[p, sl] = (w0 * r0[p, sl] + w1 * r1[p, sl]
                                 + w2 * r2[p, sl])
            return 0

        lax.fori_loop(0, pc // 16, group, 0)
        pltpu.async_copy(ov, out_hbm.at[b, pl.ds(start + c * pc, pc)],
                         osems[buf])

    fire(0, 0)

    def body(t, _):
        c0 = 2 * t
        fire(c0 + 1, 1)
        wait_gathers(c0, 0)
        compute(c0, 0, t)

        @pl.when(c0 + 2 < nchunks)
        def _():
            fire(c0 + 2, 0)

        wait_gathers(c0 + 1, 1)
        compute(c0 + 1, 1, t)
        return 0

    lax.fori_loop(0, nchunks // 2, body, 0)
    for buf in range(2):
        pltpu.make_async_copy(
            obufs[buf], out_hbm.at[b, pl.ds(0, pc)], osems[buf]).wait()


def _k2_body(f1_ref, it_ref, w1a_ref, w1b_ref, b1_ref, s1i_ref, q1i_ref,
             h1_ref, s1_ref, q1_ref):
    b = pl.program_id(0)
    i = pl.program_id(1)
    h1 = (jax.lax.dot_general(
              w1a_ref[...], f1_ref[0], (((1,), (0,)), ((), ())),
              preferred_element_type=jnp.float32)
          + jax.lax.dot_general(
              w1b_ref[...], it_ref[0], (((1,), (1,)), ((), ())),
              preferred_element_type=jnp.float32)
          + b1_ref[...])
    h1_ref[0] = h1

    @pl.when((b == 0) & (i == 0))
    def _():
        s1_ref[...] = s1i_ref[...]
        q1_ref[...] = q1i_ref[...]

    s1_ref[...] += jnp.sum(h1, axis=1, keepdims=True)
    q1_ref[...] += jnp.sum(h1 * h1, axis=1, keepdims=True)


def _bn_affine(s_ref, q_ref, g_ref, be_ref, count):
    mean = s_ref[...] / count
    var = q_ref[...] / count - mean * mean
    scale = g_ref[...] * jax.lax.rsqrt(var + 1e-5)
    shift = be_ref[...] - mean * scale
    return scale, shift


def _layer2(h1_ref, s1_ref, q1_ref, g1_ref, be1_ref, w2_ref, b2_ref, count):
    scale, shift = _bn_affine(s1_ref, q1_ref, g1_ref, be1_ref, count)
    a = jnp.maximum(h1_ref[0] * scale + shift, 0.0)
    return jax.lax.dot_general(
        w2_ref[...], a, (((1,), (0,)), ((), ())),
        preferred_element_type=jnp.float32) + b2_ref[...]


def _k3_body(h1_ref, s1_ref, q1_ref, g1_ref, be1_ref, w2_ref, b2_ref,
             s2i_ref, q2i_ref, s2_ref, q2_ref, *, count):
    b = pl.program_id(0)
    i = pl.program_id(1)
    h2 = _layer2(h1_ref, s1_ref, q1_ref, g1_ref, be1_ref, w2_ref, b2_ref,
                 count)

    @pl.when((b == 0) & (i == 0))
    def _():
        s2_ref[...] = s2i_ref[...]
        q2_ref[...] = q2i_ref[...]

    s2_ref[...] += jnp.sum(h2, axis=1, keepdims=True)
    q2_ref[...] += jnp.sum(h2 * h2, axis=1, keepdims=True)


def _k4_body(h1_ref, s1_ref, q1_ref, g1_ref, be1_ref, w2_ref, b2_ref,
             s2_ref, q2_ref, g2_ref, be2_ref, o_ref, *, count):
    h2 = _layer2(h1_ref, s1_ref, q1_ref, g1_ref, be1_ref, w2_ref, b2_ref,
                 count)
    scale, shift = _bn_affine(s2_ref, q2_ref, g2_ref, be2_ref, count)
    o_ref[0] = jnp.maximum(h2 * scale + shift, 0.0)


def kernel(centroids1, centroids2, features1, features2,
           W1, b1, g1, be1, W2, b2, g2, be2):
    B, _, N1 = centroids1.shape
    N2 = centroids2.shape[2]
    C1 = features1.shape[1]
    C2 = features2.shape[1]
    CO = W1.shape[0]
    TN = min(512, N1)
    NB = N1 // TN
    count = float(B * N1)
    NW = 32
    WPB = NW // B
    SPAN = N1 // WPB
    PC = 32

    b1c = b1.reshape(CO, 1)
    g1c = g1.reshape(CO, 1)
    be1c = be1.reshape(CO, 1)
    b2c = b2.reshape(CO, 1)
    g2c = g2.reshape(CO, 1)
    be2c = be2.reshape(CO, 1)
    W1a = W1[:, :C1]
    W1b = W1[:, C1:]

    grid = (B, NB)
    col_spec = pl.BlockSpec((CO, 1), lambda b, i: (0, 0))

    c1n = jnp.sum(jnp.transpose(centroids1, (0, 2, 1)) ** 2,
                  axis=-1).reshape(B, 1, N1)
    c2n = jnp.sum(jnp.transpose(centroids2, (0, 2, 1)) ** 2,
                  axis=-1).reshape(B, N2, 1)
    table = jnp.transpose(features2, (0, 2, 1)).reshape(B * N2, C2)

    NSPLIT = 4
    B2 = B // NSPLIT
    WPB2 = NW // B2
    SPAN2 = N1 // WPB2
    grid_h = (B2, NB)
    mesh = plsc.VectorSubcoreMesh(core_axis_name="c", subcore_axis_name="s")

    idxw = []
    for h in range(NSPLIT):
        sl = slice(h * B2, (h + 1) * B2)
        idxw.append(pl.pallas_call(
            functools.partial(_k1_body, n2=N2, tn=TN, boffs=h * B2),
            grid=grid_h,
            in_specs=[
                pl.BlockSpec((1, 1, TN), lambda b, i: (b, 0, i)),
                pl.BlockSpec((1, N2, 1), lambda b, i: (b, 0, 0)),
                pl.BlockSpec((1, 3, TN), lambda b, i: (b, 0, i)),
                pl.BlockSpec((1, 3, N2), lambda b, i: (b, 0, 0)),
            ],
            out_specs=[
                pl.BlockSpec((1, 3, TN), lambda b, i: (b, 0, i)),
                pl.BlockSpec((1, 3, TN), lambda b, i: (b, 0, i)),
            ],
            out_shape=[
                jax.ShapeDtypeStruct((B2, 3, N1), jnp.int32),
                jax.ShapeDtypeStruct((B2, 3, N1), jnp.float32),
            ],
        )(c1n[sl], c2n[sl], centroids1[sl], centroids2[sl]))

    interps = []
    for h in range(NSPLIT):
        idxg, wts = idxw[h]
        interps.append(functools.partial(
            pl.kernel,
            mesh=mesh,
            out_type=jax.ShapeDtypeStruct((B2, N1, C2), jnp.float32),
            scratch_types=(
                [pltpu.VMEM((3, SPAN2), jnp.int32),
                 pltpu.VMEM((3, SPAN2), jnp.float32)]
                + [pltpu.VMEM((PC, C2), jnp.float32) for _ in range(8)]
                + [pltpu.SemaphoreType.DMA for _ in range(8)]
            ),
        )(functools.partial(_sc_interp_body, n1=N1, c2=C2, pc=PC, wpb=WPB2)
          )(table, idxg, wts))

    zcol = jnp.zeros((CO, 1), jnp.float32)
    s1, q1 = zcol, zcol
    h1pres = []
    for h in range(NSPLIT):
        sl = slice(h * B2, (h + 1) * B2)
        h1pre, s1, q1 = pl.pallas_call(
            _k2_body,
            grid=grid_h,
            in_specs=[
                pl.BlockSpec((1, C1, TN), lambda b, i: (b, 0, i)),
                pl.BlockSpec((1, TN, C2), lambda b, i: (b, i, 0)),
                pl.BlockSpec((CO, C1), lambda b, i: (0, 0)),
                pl.BlockSpec((CO, C2), lambda b, i: (0, 0)),
                col_spec, col_spec, col_spec,
            ],
            out_specs=[
                pl.BlockSpec((1, CO, TN), lambda b, i: (b, 0, i)),
                col_spec,
                col_spec,
            ],
            out_shape=[
                jax.ShapeDtypeStruct((B2, CO, N1), jnp.float32),
                jax.ShapeDtypeStruct((CO, 1), jnp.float32),
                jax.ShapeDtypeStruct((CO, 1), jnp.float32),
            ],
        )(features1[sl], interps[h], W1a, W1b, b1c, s1, q1)
        h1pres.append(h1pre)

    TM = min(1024, N1)
    grid2h = (B2, N1 // TM)
    blk = pl.BlockSpec((1, CO, TM), lambda b, i: (b, 0, i))
    w2_spec = pl.BlockSpec((CO, CO), lambda b, i: (0, 0))

    s2, q2 = zcol, zcol
    for h in range(NSPLIT):
        s2, q2 = pl.pallas_call(
            functools.partial(_k3_body, count=count),
            grid=grid2h,
            in_specs=[blk, col_spec, col_spec, col_spec, col_spec, w2_spec,
                      col_spec, col_spec, col_spec],
            out_specs=[col_spec, col_spec],
            out_shape=[
                jax.ShapeDtypeStruct((CO, 1), jnp.float32),
                jax.ShapeDtypeStruct((CO, 1), jnp.float32),
            ],
        )(h1pres[h], s1, q1, g1c, be1c, W2, b2c, s2, q2)

    outs = []
    for h in range(NSPLIT):
        outs.append(pl.pallas_call(
            functools.partial(_k4_body, count=count),
            grid=grid2h,
            in_specs=[blk, col_spec, col_spec, col_spec, col_spec, w2_spec,
                      col_spec, col_spec, col_spec, col_spec, col_spec],
            out_specs=blk,
            out_shape=jax.ShapeDtypeStruct((B2, CO, N1), jnp.float32),
        )(h1pres[h], s1, q1, g1c, be1c, W2, b2c, s2, q2, g2c, be2c))

    return jnp.concatenate(outs, axis=0)
